# baseline (speedup 1.0000x reference)
import functools

import jax
import jax.numpy as jnp
from jax import lax
from jax.experimental import pallas as pl
from jax.experimental.pallas import tpu as pltpu
from jax.experimental.pallas import tpu_sc as plsc

N = 10000
E = 160000
D = 256
H1 = 16
H2 = 32
OUT = 16

NC = 2
NS = 16
NW = NC * NS
EW = E // NW
CH = 128
NCHUNK = 40
EWP = NCHUNK * CH
NPAD = 10112
RPT = NPAD // NS
K = 8
NGRP = NCHUNK // K

_f32 = jnp.float32


def _dot_t(x, w):
    return lax.dot_general(x, w, (((1,), (1,)), ((), ())),
                           preferred_element_type=_f32)


def _tc1_body(xu, xi, w_ag_i, w_self_u, w_ag_u, w_self_i, out):
    xu_b = xu[...]
    xi_b = xi[...]
    out[0] = _dot_t(xu_b, w_ag_i[...])
    out[1] = _dot_t(xi_b, w_ag_u[...])
    out[2] = _dot_t(xi_b, w_self_i[...])
    out[3] = _dot_t(xu_b, w_self_u[...])


def _tc1(xu, xi, w_ag_i, w_self_u, w_ag_u, w_self_i):
    blk = NPAD // 8
    x_spec = pl.BlockSpec((blk, D), lambda i: (i, 0))
    w_spec = pl.BlockSpec((H1, D), lambda i: (0, 0))
    o_spec = pl.BlockSpec((4, blk, H1), lambda i: (0, i, 0))
    return pl.pallas_call(
        _tc1_body,
        grid=(8,),
        in_specs=[x_spec, x_spec, w_spec, w_spec, w_spec, w_spec],
        out_specs=o_spec,
        out_shape=jax.ShapeDtypeStruct((4, NPAD, H1), _f32),
    )(xu, xi, w_ag_i, w_self_u, w_ag_u, w_self_i)


def _sc_agg_body(with_counts, *refs):
    n_in = 7 if with_counts else 6
    n_acc = 4 if with_counts else 2
    if with_counts:
        (tabs, src_a, dst_a, src_b, dst_b, zrows, orows) = refs[:n_in]
    else:
        (tabs, src_a, dst_a, src_b, dst_b, zrows) = refs[:n_in]
    tab_a = tabs.at[0]
    tab_b = tabs.at[1]
    outs = refs[n_in:n_in + 1]
    scr = refs[n_in + 1:]
    if with_counts:
        (src_v, dst_v, rows_v, ones_v, zv, sem_g, sem_s, sem_o,
         acc_sa, acc_sb, acc_ca, acc_cb, tab_sa, tab_sb) = scr
        accs = (acc_sa, acc_sb, acc_ca, acc_cb)
    else:
        (src_v, dst_v, rows_v, zv, sem_g, sem_s, acc_sa, acc_sb,
         tab_sa, tab_sb) = scr
        accs = (acc_sa, acc_sb)

    c = lax.axis_index("c")
    s = lax.axis_index("s")
    wid = c * NS + s

    sl = pl.ds(s * RPT, RPT)
    pltpu.sync_copy(tab_a.at[sl], tab_sa.at[sl])
    pltpu.sync_copy(tab_b.at[sl], tab_sb.at[sl])

    pltpu.sync_copy(zrows, zv)
    for acc in accs:
        pltpu.sync_copy(zv, acc.at[sl])
    if with_counts:
        pltpu.sync_copy(orows, ones_v)
    plsc.subcore_barrier()

    rels = ((tab_sa, tab_a, src_a, dst_a, 0), (tab_sb, tab_b, src_b, dst_b, 1))
    for tab_s, tab_h, srcH, dstH, r in rels:
        pltpu.sync_copy(srcH.at[wid], src_v)
        pltpu.sync_copy(dstH.at[wid], dst_v)
        acc_s = accs[r]
        acc_c = accs[2 + r] if with_counts else None

        def scat_wait(b, acc_s=acc_s):
            pltpu.make_async_copy(
                rows_v.at[b], acc_s.at[dst_v.at[0]], sem_s.at[b]).wait()

        def run_ring(tab, acc_s=acc_s, acc_c=acc_c, scat_wait=scat_wait):
            def group(g, _):
                gd = []
                for b in range(K):
                    @pl.when(g > 0)
                    def _(b=b):
                        scat_wait(b)
                    gd.append(pltpu.async_copy(
                        tab.at[src_v.at[g * K + b]], rows_v.at[b],
                        sem_g.at[b]))
                for b in range(K):
                    gd[b].wait()
                    pltpu.async_copy(rows_v.at[b],
                                     acc_s.at[dst_v.at[g * K + b]],
                                     sem_s.at[b], add=True)
                    if with_counts:
                        pltpu.async_copy(ones_v,
                                         acc_c.at[dst_v.at[g * K + b]],
                                         sem_o, add=True)
                return 0
            lax.fori_loop(0, NGRP, group, 0)

        @pl.when(s % 2 == 0)
        def _():
            run_ring(tab_s)

        @pl.when(s % 2 == 1)
        def _():
            run_ring(tab_h)

        for b in range(K):
            scat_wait(b)
        if with_counts:
            def drain(j, _, acc_c=acc_c):
                pltpu.make_async_copy(
                    ones_v, acc_c.at[dst_v.at[0]], sem_o).wait()
                return 0
            lax.fori_loop(0, NCHUNK, drain, 0)

    plsc.subcore_barrier()
    out = outs[0]
    for k, acc in enumerate(accs):
        pltpu.sync_copy(acc.at[pl.ds(s * RPT, RPT)],
                        out.at[c, k, pl.ds(s * RPT, RPT)])


def _make_sc_agg(with_counts):
    n_out = 4 if with_counts else 2
    out_t = jax.ShapeDtypeStruct((NC, n_out, NPAD, H1), _f32)
    scratch = [
        pltpu.VMEM((NCHUNK, CH), jnp.int32),
        pltpu.VMEM((NCHUNK, CH), jnp.int32),
        pltpu.VMEM((K, CH, H1), _f32),
    ]
    if with_counts:
        scratch.append(pltpu.VMEM((CH, H1), _f32))
    scratch += [
        pltpu.VMEM((RPT, H1), _f32),
        pltpu.SemaphoreType.DMA((K,)),
        pltpu.SemaphoreType.DMA((K,)),
    ]
    if with_counts:
        scratch.append(pltpu.SemaphoreType.DMA)
    scratch += [pltpu.VMEM_SHARED((NPAD, H1), _f32)] * (n_out + 2)
    mesh = plsc.VectorSubcoreMesh(core_axis_name="c", subcore_axis_name="s")
    return pl.kernel(
        functools.partial(_sc_agg_body, with_counts),
        out_type=out_t,
        mesh=mesh,
        scratch_types=scratch,
        compiler_params=pltpu.CompilerParams(use_tc_tiling_on_sc=False),
    )


def _prep_edges(ei):
    src = ei[0].reshape(NW, EW)
    dst = ei[1].reshape(NW, EW)
    pad = EWP - EW
    src = jnp.pad(src, ((0, 0), (0, pad)))
    dst = jnp.pad(dst, ((0, 0), (0, pad)), constant_values=N)
    return src.reshape(NW, NCHUNK, CH), dst.reshape(NW, NCHUNK, CH)


def _tc2_body(pk, proj, b1r, b1b, h2, rcpp):
    ri = 1.0 / jnp.maximum(pk[0, 2] + pk[1, 2], 1.0)
    ru = 1.0 / jnp.maximum(pk[0, 3] + pk[1, 3], 1.0)
    rcpp[0] = ru
    rcpp[1] = ri
    h2[1] = jnp.maximum((pk[0, 0] + pk[1, 0]) * ri + proj[0] + b1r[...], 0.0)
    h2[0] = jnp.maximum((pk[0, 1] + pk[1, 1]) * ru + proj[1] + b1b[...], 0.0)


def _tc2(pk, proj, b1r, b1b):
    blk = NPAD // 8
    p_spec = pl.BlockSpec((NC, 4, blk, H1), lambda i: (0, 0, i, 0))
    pj_spec = pl.BlockSpec((2, blk, H1), lambda i: (1, i, 0))
    b_spec = pl.BlockSpec((1, H1), lambda i: (0, 0))
    h_spec = pl.BlockSpec((2, blk, H1), lambda i: (0, i, 0))
    return pl.pallas_call(
        _tc2_body,
        grid=(8,),
        in_specs=[p_spec, pj_spec, b_spec, b_spec],
        out_specs=[h_spec, h_spec],
        out_shape=[jax.ShapeDtypeStruct((2, NPAD, H1), _f32),
                   jax.ShapeDtypeStruct((2, NPAD, H1), _f32)],
    )(pk, proj, b1r, b1b)


def _tc3_body(s2p, rcpp, h2,
              w2b_l, w2b_r, b2b_t, w2r_l, w2r_r, b2r_t,
              wu, bu_t, wi, bi_t, out):
    def head(wl, wr, b2, wh, bh):
        mean2 = (s2p[0, 0] + s2p[1, 0]) * rcpp[0]
        g = _dot_t(mean2, wl[...]) + _dot_t(h2[0], wr[...]) + b2[...]
        logit = _dot_t(g, wh[...]) + bh[...]
        m = jnp.max(logit, axis=1, keepdims=True)
        e = jnp.exp(logit - m)
        lse = m + jnp.log(jnp.sum(e, axis=1, keepdims=True))
        return logit - lse

    t = pl.program_id(0)

    @pl.when(t == 0)
    def _():
        out[...] = head(w2b_l, w2b_r, b2b_t, wu, bu_t)

    @pl.when(t == 1)
    def _():
        out[...] = head(w2r_l, w2r_r, b2r_t, wi, bi_t)


def _tc3(s2p, rcpp, h2,
         w2b_l, w2b_r, b2b_t, w2r_l, w2r_r, b2r_t, wu, bu_t, wi, bi_t):
    blk = 2000
    p_spec = pl.BlockSpec((NC, 1, blk, H1), lambda t, i: (0, 1 - t, i, 0))
    h_spec = pl.BlockSpec((1, blk, H1), lambda t, i: (t, i, 0))
    w2_spec = pl.BlockSpec((H2, H1), lambda t, i: (0, 0))
    bh2_spec = pl.BlockSpec((1, H2), lambda t, i: (0, 0))
    wh_spec = pl.BlockSpec((OUT, H2), lambda t, i: (0, 0))
    bo_spec = pl.BlockSpec((1, OUT), lambda t, i: (0, 0))
    o_spec = pl.BlockSpec((blk, OUT), lambda t, i: (t * (N // blk) + i, 0))
    return pl.pallas_call(
        _tc3_body,
        grid=(2, N // blk),
        in_specs=[p_spec, h_spec, h_spec,
                  w2_spec, w2_spec, bh2_spec, w2_spec, w2_spec, bh2_spec,
                  wh_spec, bo_spec, wh_spec, bo_spec],
        out_specs=o_spec,
        out_shape=jax.ShapeDtypeStruct((2 * N, OUT), _f32),
    )(s2p, rcpp, h2,
      w2b_l, w2b_r, b2b_t, w2r_l, w2r_r, b2r_t, wu, bu_t, wi, bi_t)


def kernel(x_user, x_item, edge_index_rates, edge_index_ratedby,
           W1r_l, b1r, W1r_r, W1b_l, b1b, W1b_r,
           W2r_l, b2r, W2r_r, W2b_l, b2b, W2b_r,
           Wu, bu, Wi, bi):
    src_r, dst_r = _prep_edges(edge_index_rates)
    src_b, dst_b = _prep_edges(edge_index_ratedby)
    zrows = jnp.zeros((RPT, H1), _f32)
    orows = jnp.ones((CH, H1), _f32)

    proj = _tc1(x_user, x_item, W1r_l, W1b_r, W1b_l, W1r_r)

    pk = _make_sc_agg(True)(proj, src_r, dst_r, src_b, dst_b, zrows, orows)

    h2, rcpp = _tc2(pk, proj, b1r.reshape(1, H1), b1b.reshape(1, H1))

    s2p = _make_sc_agg(False)(h2, src_r, dst_r, src_b, dst_b, zrows)

    return _tc3(
        s2p, rcpp, h2,
        W2b_l, W2b_r, b2b.reshape(1, H2),
        W2r_l, W2r_r, b2r.reshape(1, H2),
        Wu, bu.reshape(1, OUT), Wi, bi.reshape(1, OUT))


# speedup vs baseline: 1.0910x; 1.0910x over previous
import functools

import jax
import jax.numpy as jnp
from jax import lax
from jax.experimental import pallas as pl
from jax.experimental.pallas import tpu as pltpu
from jax.experimental.pallas import tpu_sc as plsc

N = 10000
E = 160000
D = 256
H1 = 16
H2 = 32
OUT = 16

NC = 2
NS = 16
NW = NC * NS
EW = E // NW
CH = 128
NCHUNK = 40
EWP = NCHUNK * CH
NPAD = 10112
RPT = NPAD // NS
K = 8
NGRP = NCHUNK // K

_f32 = jnp.float32


def _dot_t(x, w):
    return lax.dot_general(x, w, (((1,), (1,)), ((), ())),
                           preferred_element_type=_f32)


def _tc1_body(xu, xi, w_ag_i, w_self_u, w_ag_u, w_self_i, out):
    xu_b = xu[...]
    xi_b = xi[...]
    out[0] = _dot_t(xu_b, w_ag_i[...])
    out[1] = _dot_t(xi_b, w_ag_u[...])
    out[2] = _dot_t(xi_b, w_self_i[...])
    out[3] = _dot_t(xu_b, w_self_u[...])


def _tc1(xu, xi, w_ag_i, w_self_u, w_ag_u, w_self_i):
    blk = NPAD // 8
    x_spec = pl.BlockSpec((blk, D), lambda i: (i, 0))
    w_spec = pl.BlockSpec((H1, D), lambda i: (0, 0))
    o_spec = pl.BlockSpec((4, blk, H1), lambda i: (0, i, 0))
    return pl.pallas_call(
        _tc1_body,
        grid=(8,),
        in_specs=[x_spec, x_spec, w_spec, w_spec, w_spec, w_spec],
        out_specs=o_spec,
        out_shape=jax.ShapeDtypeStruct((4, NPAD, H1), _f32),
    )(xu, xi, w_ag_i, w_self_u, w_ag_u, w_self_i)


def _sc_agg_body(with_counts, *refs):
    n_in = 7 if with_counts else 6
    n_acc = 4 if with_counts else 2
    if with_counts:
        (tabs, src_a, dst_a, src_b, dst_b, zrows, orows) = refs[:n_in]
    else:
        (tabs, src_a, dst_a, src_b, dst_b, zrows) = refs[:n_in]
    tab_a = tabs.at[0]
    tab_b = tabs.at[1]
    outs = refs[n_in:n_in + 1]
    scr = refs[n_in + 1:]
    if with_counts:
        (src_v, dst_v, rows_v, ones_v, zv, sem_g, sem_s, sem_o,
         acc_sa, acc_sb, acc_ca, acc_cb, tab_sa, tab_sb) = scr
        accs = (acc_sa, acc_sb, acc_ca, acc_cb)
    else:
        (src_v, dst_v, rows_v, zv, sem_g, sem_s, acc_sa, acc_sb,
         tab_sa, tab_sb) = scr
        accs = (acc_sa, acc_sb)

    c = lax.axis_index("c")
    s = lax.axis_index("s")
    wid = c * NS + s

    sl = pl.ds(s * RPT, RPT)
    pltpu.sync_copy(zrows, zv)
    start = [
        pltpu.async_copy(tab_a.at[sl], tab_sa.at[sl], sem_g.at[0]),
        pltpu.async_copy(tab_b.at[sl], tab_sb.at[sl], sem_g.at[1]),
        pltpu.async_copy(src_a.at[wid], src_v.at[0], sem_s.at[0]),
        pltpu.async_copy(dst_a.at[wid], dst_v.at[0], sem_s.at[1]),
        pltpu.async_copy(src_b.at[wid], src_v.at[1], sem_s.at[2]),
        pltpu.async_copy(dst_b.at[wid], dst_v.at[1], sem_s.at[3]),
    ]
    for k, acc in enumerate(accs):
        start.append(pltpu.async_copy(zv, acc.at[sl], sem_g.at[2 + k]))
    if with_counts:
        pltpu.sync_copy(orows, ones_v)
    for d in start:
        d.wait()
    plsc.subcore_barrier()

    rels = ((tab_sa, 0), (tab_sb, 1))
    for tab, r in rels:
        acc_s = accs[r]
        acc_c = accs[2 + r] if with_counts else None

        def scat_wait(b, acc_s=acc_s, r=r):
            pltpu.make_async_copy(
                rows_v.at[b], acc_s.at[dst_v.at[r, 0]], sem_s.at[b]).wait()

        def group(g, _, tab=tab, acc_s=acc_s, acc_c=acc_c, r=r,
                  scat_wait=scat_wait):
            gd = []
            for b in range(K):
                @pl.when(g > 0)
                def _(b=b):
                    scat_wait(b)
                gd.append(pltpu.async_copy(
                    tab.at[src_v.at[r, g * K + b]], rows_v.at[b],
                    sem_g.at[b]))
            for b in range(K):
                gd[b].wait()
                pltpu.async_copy(rows_v.at[b],
                                 acc_s.at[dst_v.at[r, g * K + b]],
                                 sem_s.at[b], add=True)
                if with_counts:
                    pltpu.async_copy(ones_v,
                                     acc_c.at[dst_v.at[r, g * K + b]],
                                     sem_o, add=True)
            return 0

        lax.fori_loop(0, NGRP, group, 0)

        for b in range(K):
            scat_wait(b)

    if with_counts:
        def drain(j, _):
            pltpu.make_async_copy(
                ones_v, accs[2].at[dst_v.at[0]], sem_o).wait()
            return 0
        lax.fori_loop(0, 2 * NCHUNK, drain, 0)

    plsc.subcore_barrier()
    out = outs[0]
    wb = []
    for k, acc in enumerate(accs):
        wb.append(pltpu.async_copy(
            acc.at[sl], out.at[c, k, sl], sem_g.at[k]))
    for d in wb:
        d.wait()


def _make_sc_agg(with_counts):
    n_out = 4 if with_counts else 2
    out_t = jax.ShapeDtypeStruct((NC, n_out, NPAD, H1), _f32)
    scratch = [
        pltpu.VMEM((2, NCHUNK, CH), jnp.int32),
        pltpu.VMEM((2, NCHUNK, CH), jnp.int32),
        pltpu.VMEM((K, CH, H1), _f32),
    ]
    if with_counts:
        scratch.append(pltpu.VMEM((CH, H1), _f32))
    scratch += [
        pltpu.VMEM((RPT, H1), _f32),
        pltpu.SemaphoreType.DMA((K,)),
        pltpu.SemaphoreType.DMA((K,)),
    ]
    if with_counts:
        scratch.append(pltpu.SemaphoreType.DMA)
    scratch += [pltpu.VMEM_SHARED((NPAD, H1), _f32)] * (n_out + 2)
    mesh = plsc.VectorSubcoreMesh(core_axis_name="c", subcore_axis_name="s")
    return pl.kernel(
        functools.partial(_sc_agg_body, with_counts),
        out_type=out_t,
        mesh=mesh,
        scratch_types=scratch,
        compiler_params=pltpu.CompilerParams(use_tc_tiling_on_sc=False),
    )


def _prep_edges(ei):
    src = ei[0].reshape(NW, EW)
    dst = ei[1].reshape(NW, EW)
    pad = EWP - EW
    src = jnp.pad(src, ((0, 0), (0, pad)))
    dst = jnp.pad(dst, ((0, 0), (0, pad)), constant_values=N)
    return src.reshape(NW, NCHUNK, CH), dst.reshape(NW, NCHUNK, CH)


def _tc2_body(pk, proj, b1r, b1b, h2, rcpp):
    ri = 1.0 / jnp.maximum(pk[0, 2] + pk[1, 2], 1.0)
    ru = 1.0 / jnp.maximum(pk[0, 3] + pk[1, 3], 1.0)
    rcpp[0] = ru
    rcpp[1] = ri
    h2[1] = jnp.maximum((pk[0, 0] + pk[1, 0]) * ri + proj[0] + b1r[...], 0.0)
    h2[0] = jnp.maximum((pk[0, 1] + pk[1, 1]) * ru + proj[1] + b1b[...], 0.0)


def _tc2(pk, proj, b1r, b1b):
    blk = NPAD // 8
    p_spec = pl.BlockSpec((NC, 4, blk, H1), lambda i: (0, 0, i, 0))
    pj_spec = pl.BlockSpec((2, blk, H1), lambda i: (1, i, 0))
    b_spec = pl.BlockSpec((1, H1), lambda i: (0, 0))
    h_spec = pl.BlockSpec((2, blk, H1), lambda i: (0, i, 0))
    return pl.pallas_call(
        _tc2_body,
        grid=(8,),
        in_specs=[p_spec, pj_spec, b_spec, b_spec],
        out_specs=[h_spec, h_spec],
        out_shape=[jax.ShapeDtypeStruct((2, NPAD, H1), _f32),
                   jax.ShapeDtypeStruct((2, NPAD, H1), _f32)],
    )(pk, proj, b1r, b1b)


def _tc3_body(s2p, rcpp, h2,
              w2b_l, w2b_r, b2b_t, w2r_l, w2r_r, b2r_t,
              wu, bu_t, wi, bi_t, out):
    def head(wl, wr, b2, wh, bh):
        mean2 = (s2p[0, 0] + s2p[1, 0]) * rcpp[0]
        g = _dot_t(mean2, wl[...]) + _dot_t(h2[0], wr[...]) + b2[...]
        logit = _dot_t(g, wh[...]) + bh[...]
        m = jnp.max(logit, axis=1, keepdims=True)
        e = jnp.exp(logit - m)
        lse = m + jnp.log(jnp.sum(e, axis=1, keepdims=True))
        return logit - lse

    t = pl.program_id(0)

    @pl.when(t == 0)
    def _():
        out[...] = head(w2b_l, w2b_r, b2b_t, wu, bu_t)

    @pl.when(t == 1)
    def _():
        out[...] = head(w2r_l, w2r_r, b2r_t, wi, bi_t)


def _tc3(s2p, rcpp, h2,
         w2b_l, w2b_r, b2b_t, w2r_l, w2r_r, b2r_t, wu, bu_t, wi, bi_t):
    blk = 2000
    p_spec = pl.BlockSpec((NC, 1, blk, H1), lambda t, i: (0, 1 - t, i, 0))
    h_spec = pl.BlockSpec((1, blk, H1), lambda t, i: (t, i, 0))
    w2_spec = pl.BlockSpec((H2, H1), lambda t, i: (0, 0))
    bh2_spec = pl.BlockSpec((1, H2), lambda t, i: (0, 0))
    wh_spec = pl.BlockSpec((OUT, H2), lambda t, i: (0, 0))
    bo_spec = pl.BlockSpec((1, OUT), lambda t, i: (0, 0))
    o_spec = pl.BlockSpec((blk, OUT), lambda t, i: (t * (N // blk) + i, 0))
    return pl.pallas_call(
        _tc3_body,
        grid=(2, N // blk),
        in_specs=[p_spec, h_spec, h_spec,
                  w2_spec, w2_spec, bh2_spec, w2_spec, w2_spec, bh2_spec,
                  wh_spec, bo_spec, wh_spec, bo_spec],
        out_specs=o_spec,
        out_shape=jax.ShapeDtypeStruct((2 * N, OUT), _f32),
    )(s2p, rcpp, h2,
      w2b_l, w2b_r, b2b_t, w2r_l, w2r_r, b2r_t, wu, bu_t, wi, bi_t)


def kernel(x_user, x_item, edge_index_rates, edge_index_ratedby,
           W1r_l, b1r, W1r_r, W1b_l, b1b, W1b_r,
           W2r_l, b2r, W2r_r, W2b_l, b2b, W2b_r,
           Wu, bu, Wi, bi):
    src_r, dst_r = _prep_edges(edge_index_rates)
    src_b, dst_b = _prep_edges(edge_index_ratedby)
    zrows = jnp.zeros((RPT, H1), _f32)
    orows = jnp.ones((CH, H1), _f32)

    proj = _tc1(x_user, x_item, W1r_l, W1b_r, W1b_l, W1r_r)

    pk = _make_sc_agg(True)(proj, src_r, dst_r, src_b, dst_b, zrows, orows)

    h2, rcpp = _tc2(pk, proj, b1r.reshape(1, H1), b1b.reshape(1, H1))

    s2p = _make_sc_agg(False)(h2, src_r, dst_r, src_b, dst_b, zrows)

    return _tc3(
        s2p, rcpp, h2,
        W2b_l, W2b_r, b2b.reshape(1, H2),
        W2r_l, W2r_r, b2r.reshape(1, H2),
        Wu, bu.reshape(1, OUT), Wi, bi.reshape(1, OUT))
